# baseline (speedup 1.0000x reference)
import functools

import jax
import jax.numpy as jnp
from jax import lax
from jax.experimental import pallas as pl
from jax.experimental.pallas import tpu as pltpu
from jax.experimental.pallas import tpu_sc as plsc

_NUM_EXPERTS = 16
_TOP_K = 2
_TM = 256

_SC_NC = 2
_SC_NS = 16
_SC_NW = _SC_NC * _SC_NS


def _sc_gather_rows(table, idx, chunk):
    n_rows, d = table.shape
    b = idx.shape[0]
    per_w = b // _SC_NW
    assert per_w % chunk == 0 and b % (8 * _SC_NW) == 0
    n_chunks = per_w // chunk
    mesh = plsc.VectorSubcoreMesh(core_axis_name="c", subcore_axis_name="s")

    @functools.partial(
        pl.kernel,
        mesh=mesh,
        out_type=jax.ShapeDtypeStruct((b, d), table.dtype),
        scratch_types=[
            pltpu.VMEM((per_w,), jnp.int32),
            pltpu.VMEM((chunk, d), table.dtype),
            pltpu.SemaphoreType.DMA,
        ],
    )
    def gather_kernel(table_hbm, idx_hbm, out_hbm, idx_v, rows_v, sem):
        wid = lax.axis_index("s") * _SC_NC + lax.axis_index("c")
        base = wid * per_w
        pltpu.sync_copy(idx_hbm.at[pl.ds(base, per_w)], idx_v)
        for j in range(n_chunks):
            pltpu.async_copy(
                table_hbm.at[idx_v.at[pl.ds(j * chunk, chunk)]], rows_v, sem
            ).wait()
            pltpu.sync_copy(rows_v, out_hbm.at[pl.ds(base + j * chunk, chunk)])

    return gather_kernel(table, idx)


def _sc_combine_pairs(table, idx_even, idx_odd, chunk):
    n_rows, d = table.shape
    t = idx_even.shape[0]
    per_w = t // _SC_NW
    assert per_w % chunk == 0 and t % (8 * _SC_NW) == 0
    n_chunks = per_w // chunk
    mesh = plsc.VectorSubcoreMesh(core_axis_name="c", subcore_axis_name="s")

    @functools.partial(
        pl.kernel,
        mesh=mesh,
        out_type=jax.ShapeDtypeStruct((t, d), table.dtype),
        scratch_types=[
            pltpu.VMEM((per_w,), jnp.int32),
            pltpu.VMEM((per_w,), jnp.int32),
            pltpu.VMEM((chunk, d), table.dtype),
            pltpu.VMEM((chunk, d), table.dtype),
            pltpu.SemaphoreType.DMA,
        ],
    )
    def combine_kernel(table_hbm, ie_hbm, io_hbm, out_hbm, ie_v, io_v, a_v, b_v, sem):
        wid = lax.axis_index("s") * _SC_NC + lax.axis_index("c")
        base = wid * per_w
        pltpu.sync_copy(ie_hbm.at[pl.ds(base, per_w)], ie_v)
        pltpu.sync_copy(io_hbm.at[pl.ds(base, per_w)], io_v)
        for c in range(n_chunks):
            pltpu.async_copy(
                table_hbm.at[ie_v.at[pl.ds(c * chunk, chunk)]], a_v, sem
            ).wait()
            pltpu.async_copy(
                table_hbm.at[io_v.at[pl.ds(c * chunk, chunk)]], b_v, sem
            ).wait()

            def row_add(r, _):
                for kk in range(d // 16):
                    sl = pl.ds(kk * 16, 16)
                    a_v[r, sl] = a_v[r, sl] + b_v[r, sl]
                return _

            lax.fori_loop(0, chunk, row_add, 0)
            pltpu.sync_copy(a_v, out_hbm.at[pl.ds(base + c * chunk, chunk)])

    return combine_kernel(table, idx_even, idx_odd)


def _gmm_body(te_ref, tv_ref, hs_ref, w_ref, wg_ref, wu_ref, wd_ref, out_ref):
    t = pl.program_id(0)

    @pl.when(tv_ref[t] == 1)
    def _():
        rows = hs_ref[...].astype(jnp.bfloat16)
        wg = wg_ref[0].astype(jnp.bfloat16)
        wu = wu_ref[0].astype(jnp.bfloat16)
        wd = wd_ref[0].astype(jnp.bfloat16)
        dn = (((1,), (1,)), ((), ()))
        g = jax.lax.dot_general(rows, wg, dn, preferred_element_type=jnp.float32)
        u = jax.lax.dot_general(rows, wu, dn, preferred_element_type=jnp.float32)
        a = (g * jax.nn.sigmoid(g)) * u
        d = jax.lax.dot_general(a.astype(jnp.bfloat16), wd, dn,
                                preferred_element_type=jnp.float32)
        out_ref[...] = d * w_ref[...]


def kernel(x, Wr, Wg, Wu, Wd):
    bsz, seq, dim = x.shape
    moe_dim = Wg.shape[1]
    h = x.reshape(-1, dim)
    T = h.shape[0]
    P = T * _TOP_K

    logits = h @ Wr.T
    scores = jax.nn.softmax(logits.astype(jnp.float32), axis=-1)
    topk_w, topk_idx = jax.lax.top_k(scores, _TOP_K)

    flat_e = topk_idx.reshape(-1).astype(jnp.int32)
    w_flat = topk_w.reshape(-1)
    onehot = (flat_e[:, None] == jnp.arange(_NUM_EXPERTS, dtype=jnp.int32)[None, :]
              ).astype(jnp.int32)
    cum_oh = jnp.cumsum(onehot, axis=0)
    counts = cum_oh[-1]
    rank = ((cum_oh - onehot) * onehot).sum(axis=1)

    padded = ((counts + _TM - 1) // _TM) * _TM
    pstart = (jnp.cumsum(padded) - padded).astype(jnp.int32)
    gstart = (jnp.cumsum(counts) - counts).astype(jnp.int32)
    dst = pstart[flat_e] + rank

    order = jnp.argsort(flat_e, stable=True)

    M_pad = P + _NUM_EXPERTS * _TM
    NT = M_pad // _TM
    slot = jnp.arange(M_pad, dtype=jnp.int32)
    cum_padded = jnp.cumsum(padded).astype(jnp.int32)
    slot_e = (slot[:, None] >= cum_padded[None, :]).astype(jnp.int32).sum(axis=1)
    slot_e = jnp.minimum(slot_e, _NUM_EXPERTS - 1)
    j = slot - pstart[slot_e]
    real = j < counts[slot_e]
    pair_idx = order[jnp.clip(gstart[slot_e] + jnp.minimum(j, counts[slot_e] - 1),
                              0, P - 1)]
    src = jnp.where(real, pair_idx // _TOP_K, slot % T).astype(jnp.int32)
    w_slot = jnp.where(real, w_flat[pair_idx], 0.0)

    hs_pad = _sc_gather_rows(h, src, 64)

    tiles = (padded // _TM).astype(jnp.int32)
    cum_tiles = jnp.cumsum(tiles)
    t_idx = jnp.arange(NT, dtype=jnp.int32)
    tile_e = jnp.searchsorted(cum_tiles, t_idx, side="right").astype(jnp.int32)
    tile_valid = (tile_e < _NUM_EXPERTS).astype(jnp.int32)
    tile_e = jnp.minimum(tile_e, _NUM_EXPERTS - 1)

    grid_spec = pltpu.PrefetchScalarGridSpec(
        num_scalar_prefetch=2,
        grid=(NT,),
        in_specs=[
            pl.BlockSpec((_TM, dim), lambda t, te, tv: (t, 0)),
            pl.BlockSpec((_TM, 1), lambda t, te, tv: (t, 0)),
            pl.BlockSpec((1, moe_dim, dim), lambda t, te, tv: (te[t], 0, 0)),
            pl.BlockSpec((1, moe_dim, dim), lambda t, te, tv: (te[t], 0, 0)),
            pl.BlockSpec((1, dim, moe_dim), lambda t, te, tv: (te[t], 0, 0)),
        ],
        out_specs=pl.BlockSpec((_TM, dim), lambda t, te, tv: (t, 0)),
    )
    out_pad = pl.pallas_call(
        _gmm_body,
        grid_spec=grid_spec,
        out_shape=jax.ShapeDtypeStruct((M_pad, dim), jnp.float32),
        compiler_params=pltpu.CompilerParams(
            dimension_semantics=("arbitrary",)),
    )(tile_e, tile_valid, hs_pad, w_slot[:, None], Wg, Wu, Wd)

    y = _sc_combine_pairs(out_pad, dst[0::2], dst[1::2], 32)
    return y.reshape(bsz, seq, dim)


# speedup vs baseline: 1.6892x; 1.6892x over previous
import functools

import jax
import jax.numpy as jnp
from jax import lax
from jax.experimental import pallas as pl
from jax.experimental.pallas import tpu as pltpu
from jax.experimental.pallas import tpu_sc as plsc

_NUM_EXPERTS = 16
_TOP_K = 2
_TM = 128

_SC_NC = 2
_SC_NS = 16
_SC_NW = _SC_NC * _SC_NS


def _router_body(h_ref, wr_ref, idx_ref, w_ref):
    logits = jax.lax.dot_general(
        h_ref[...], wr_ref[...], (((1,), (1,)), ((), ())),
        preferred_element_type=jnp.float32)
    m = jnp.max(logits, axis=1, keepdims=True)
    p = jnp.exp(logits - m)
    probs = p / jnp.sum(p, axis=1, keepdims=True)
    ids = jax.lax.broadcasted_iota(jnp.int32, probs.shape, 1)
    m1 = jnp.max(probs, axis=1, keepdims=True)
    i1 = jnp.min(jnp.where(probs == m1, ids, _NUM_EXPERTS), axis=1, keepdims=True)
    masked = jnp.where(ids == i1, -jnp.inf, probs)
    m2 = jnp.max(masked, axis=1, keepdims=True)
    i2 = jnp.min(jnp.where(masked == m2, ids, _NUM_EXPERTS), axis=1, keepdims=True)
    idx_ref[...] = jnp.concatenate([i1, i2], axis=1)
    w_ref[...] = jnp.concatenate([m1, m2], axis=1)


def _router(h, Wr):
    t, dim = h.shape
    return pl.pallas_call(
        _router_body,
        out_shape=(
            jax.ShapeDtypeStruct((t, _TOP_K), jnp.int32),
            jax.ShapeDtypeStruct((t, _TOP_K), jnp.float32),
        ),
    )(h, Wr)


def _sc_dispatch(h, dst_even, dst_odd, m_pad):
    t, d = h.shape
    per_w = t // _SC_NW
    mesh = plsc.VectorSubcoreMesh(core_axis_name="c", subcore_axis_name="s")

    @functools.partial(
        pl.kernel,
        mesh=mesh,
        out_type=jax.ShapeDtypeStruct((m_pad, d), h.dtype),
        scratch_types=[
            pltpu.VMEM((per_w,), jnp.int32),
            pltpu.VMEM((per_w,), jnp.int32),
            pltpu.VMEM((per_w, d), h.dtype),
            pltpu.SemaphoreType.DMA,
        ],
    )
    def dispatch_kernel(h_hbm, ie_hbm, io_hbm, out_hbm, ie_v, io_v, rows_v, sem):
        wid = lax.axis_index("s") * _SC_NC + lax.axis_index("c")
        base = wid * per_w
        pltpu.sync_copy(ie_hbm.at[wid], ie_v)
        pltpu.sync_copy(io_hbm.at[wid], io_v)
        pltpu.sync_copy(h_hbm.at[pl.ds(base, per_w)], rows_v)
        pltpu.async_copy(rows_v, out_hbm.at[ie_v], sem).wait()
        pltpu.async_copy(rows_v, out_hbm.at[io_v], sem).wait()

    return dispatch_kernel(h, dst_even, dst_odd)


def _sc_combine_pairs(table, idx_even, idx_odd, w_even, w_odd, chunk):
    n_rows, d = table.shape
    t = idx_even.shape[0]
    per_w = t // _SC_NW
    assert per_w % chunk == 0 and t % (8 * _SC_NW) == 0
    n_chunks = per_w // chunk
    mesh = plsc.VectorSubcoreMesh(core_axis_name="c", subcore_axis_name="s")

    @functools.partial(
        pl.kernel,
        mesh=mesh,
        out_type=jax.ShapeDtypeStruct((t, d), table.dtype),
        scratch_types=[
            pltpu.VMEM((per_w,), jnp.int32),
            pltpu.VMEM((per_w,), jnp.int32),
            pltpu.VMEM((per_w, 16), jnp.float32),
            pltpu.VMEM((per_w, 16), jnp.float32),
            pltpu.VMEM((chunk, d), table.dtype),
            pltpu.VMEM((chunk, d), table.dtype),
            pltpu.SemaphoreType.DMA,
        ],
    )
    def combine_kernel(table_hbm, ie_hbm, io_hbm, we_hbm, wo_hbm, out_hbm,
                       ie_v, io_v, we_v, wo_v, a_v, b_v, sem):
        wid = lax.axis_index("s") * _SC_NC + lax.axis_index("c")
        base = wid * per_w
        pltpu.sync_copy(ie_hbm.at[pl.ds(base, per_w)], ie_v)
        pltpu.sync_copy(io_hbm.at[pl.ds(base, per_w)], io_v)
        pltpu.sync_copy(we_hbm.at[pl.ds(base, per_w)], we_v)
        pltpu.sync_copy(wo_hbm.at[pl.ds(base, per_w)], wo_v)
        for c in range(n_chunks):
            pltpu.async_copy(
                table_hbm.at[ie_v.at[pl.ds(c * chunk, chunk)]], a_v, sem
            ).wait()
            pltpu.async_copy(
                table_hbm.at[io_v.at[pl.ds(c * chunk, chunk)]], b_v, sem
            ).wait()

            def row_add(r, carry):
                gr = c * chunk + r
                we = we_v[gr, :]
                wo = wo_v[gr, :]
                for kk in range(d // 16):
                    sl = pl.ds(kk * 16, 16)
                    a_v[r, sl] = a_v[r, sl] * we + b_v[r, sl] * wo
                return carry

            lax.fori_loop(0, chunk, row_add, 0)
            pltpu.sync_copy(a_v, out_hbm.at[pl.ds(base + c * chunk, chunk)])

    return combine_kernel(table, idx_even, idx_odd, w_even, w_odd)


def _gmm_body(te_ref, tv_ref, hs_ref, wg_ref, wu_ref, wd_ref, out_ref):
    t = pl.program_id(0)

    @pl.when(tv_ref[t] == 1)
    def _():
        rows = hs_ref[...].astype(jnp.bfloat16)
        wg = wg_ref[0].astype(jnp.bfloat16)
        wu = wu_ref[0].astype(jnp.bfloat16)
        wd = wd_ref[0].astype(jnp.bfloat16)
        dn = (((1,), (1,)), ((), ()))
        g = jax.lax.dot_general(rows, wg, dn, preferred_element_type=jnp.float32)
        u = jax.lax.dot_general(rows, wu, dn, preferred_element_type=jnp.float32)
        a = (g * jax.nn.sigmoid(g)) * u
        out_ref[...] = jax.lax.dot_general(a.astype(jnp.bfloat16), wd, dn,
                                           preferred_element_type=jnp.float32)


def kernel(x, Wr, Wg, Wu, Wd):
    bsz, seq, dim = x.shape
    moe_dim = Wg.shape[1]
    h = x.reshape(-1, dim)
    T = h.shape[0]
    P = T * _TOP_K

    topk_idx, topk_w = _router(h, Wr)

    flat_e = topk_idx.reshape(-1)
    onehot = (flat_e[:, None] == jnp.arange(_NUM_EXPERTS, dtype=jnp.int32)[None, :]
              ).astype(jnp.int32)
    cum_oh = jnp.cumsum(onehot, axis=0)
    counts = cum_oh[-1]
    rank = ((cum_oh - onehot) * onehot).sum(axis=1)

    padded = ((counts + _TM - 1) // _TM) * _TM
    pstart = (jnp.cumsum(padded) - padded).astype(jnp.int32)
    dst = pstart[flat_e] + rank

    M_pad = P + _NUM_EXPERTS * _TM
    NT = M_pad // _TM

    dst2 = dst.reshape(T, _TOP_K)
    dst_even = dst2[:, 0].reshape(_SC_NW, T // _SC_NW)
    dst_odd = dst2[:, 1].reshape(_SC_NW, T // _SC_NW)

    hs_pad = _sc_dispatch(h, dst_even, dst_odd, M_pad)

    tiles = (padded // _TM).astype(jnp.int32)
    cum_tiles = jnp.cumsum(tiles)
    t_idx = jnp.arange(NT, dtype=jnp.int32)
    tile_e = jnp.searchsorted(cum_tiles, t_idx, side="right").astype(jnp.int32)
    tile_valid = (tile_e < _NUM_EXPERTS).astype(jnp.int32)
    tile_e = jnp.minimum(tile_e, _NUM_EXPERTS - 1)

    grid_spec = pltpu.PrefetchScalarGridSpec(
        num_scalar_prefetch=2,
        grid=(NT,),
        in_specs=[
            pl.BlockSpec((_TM, dim), lambda t, te, tv: (t, 0)),
            pl.BlockSpec((1, moe_dim, dim), lambda t, te, tv: (te[t], 0, 0)),
            pl.BlockSpec((1, moe_dim, dim), lambda t, te, tv: (te[t], 0, 0)),
            pl.BlockSpec((1, dim, moe_dim), lambda t, te, tv: (te[t], 0, 0)),
        ],
        out_specs=pl.BlockSpec((_TM, dim), lambda t, te, tv: (t, 0)),
    )
    out_pad = pl.pallas_call(
        _gmm_body,
        grid_spec=grid_spec,
        out_shape=jax.ShapeDtypeStruct((M_pad, dim), jnp.float32),
        compiler_params=pltpu.CompilerParams(
            dimension_semantics=("arbitrary",)),
    )(tile_e, tile_valid, hs_pad, Wg, Wu, Wd)

    we_exp = jnp.broadcast_to(topk_w[:, 0:1], (T, 16))
    wo_exp = jnp.broadcast_to(topk_w[:, 1:2], (T, 16))
    y = _sc_combine_pairs(out_pad, dst2[:, 0], dst2[:, 1], we_exp, wo_exp, 32)
    return y.reshape(bsz, seq, dim)


# speedup vs baseline: 1.9894x; 1.1777x over previous
import functools

import jax
import jax.numpy as jnp
from jax import lax
from jax.experimental import pallas as pl
from jax.experimental.pallas import tpu as pltpu
from jax.experimental.pallas import tpu_sc as plsc

_NUM_EXPERTS = 16
_TOP_K = 2
_TM = 128

_SC_NC = 2
_SC_NS = 16
_SC_NW = _SC_NC * _SC_NS


def _router_body(h_ref, wr_ref, idx_ref, w_ref):
    logits = jax.lax.dot_general(
        h_ref[...], wr_ref[...], (((1,), (1,)), ((), ())),
        preferred_element_type=jnp.float32)
    m = jnp.max(logits, axis=1, keepdims=True)
    p = jnp.exp(logits - m)
    probs = p / jnp.sum(p, axis=1, keepdims=True)
    ids = jax.lax.broadcasted_iota(jnp.int32, probs.shape, 1)
    m1 = jnp.max(probs, axis=1, keepdims=True)
    i1 = jnp.min(jnp.where(probs == m1, ids, _NUM_EXPERTS), axis=1, keepdims=True)
    masked = jnp.where(ids == i1, -jnp.inf, probs)
    m2 = jnp.max(masked, axis=1, keepdims=True)
    i2 = jnp.min(jnp.where(masked == m2, ids, _NUM_EXPERTS), axis=1, keepdims=True)
    idx_ref[...] = jnp.concatenate([i1, i2], axis=1)
    w_ref[...] = jnp.concatenate([m1, m2], axis=1)


def _router(h, Wr):
    t, dim = h.shape
    return pl.pallas_call(
        _router_body,
        out_shape=(
            jax.ShapeDtypeStruct((t, _TOP_K), jnp.int32),
            jax.ShapeDtypeStruct((t, _TOP_K), jnp.float32),
        ),
    )(h, Wr)


def _sc_dispatch(h, dst_even, dst_odd, m_pad):
    t, d = h.shape
    per_w = t // _SC_NW
    mesh = plsc.VectorSubcoreMesh(core_axis_name="c", subcore_axis_name="s")

    @functools.partial(
        pl.kernel,
        mesh=mesh,
        out_type=jax.ShapeDtypeStruct((m_pad, d), h.dtype),
        scratch_types=[
            pltpu.VMEM((per_w,), jnp.int32),
            pltpu.VMEM((per_w,), jnp.int32),
            pltpu.VMEM((per_w, d), h.dtype),
            pltpu.SemaphoreType.DMA,
        ],
    )
    def dispatch_kernel(h_hbm, ie_hbm, io_hbm, out_hbm, ie_v, io_v, rows_v, sem):
        wid = lax.axis_index("s") * _SC_NC + lax.axis_index("c")
        base = wid * per_w
        pltpu.sync_copy(ie_hbm.at[wid], ie_v)
        pltpu.sync_copy(io_hbm.at[wid], io_v)
        pltpu.sync_copy(h_hbm.at[pl.ds(base, per_w)], rows_v)
        pltpu.async_copy(rows_v, out_hbm.at[ie_v], sem).wait()
        pltpu.async_copy(rows_v, out_hbm.at[io_v], sem).wait()

    return dispatch_kernel(h, dst_even, dst_odd)


def _sc_combine_pairs(table, idx_even, idx_odd, w_even, w_odd, chunk):
    n_rows, d = table.shape
    t = idx_even.shape[0]
    per_w = t // _SC_NW
    assert per_w % chunk == 0 and t % (8 * _SC_NW) == 0
    n_chunks = per_w // chunk
    mesh = plsc.VectorSubcoreMesh(core_axis_name="c", subcore_axis_name="s")

    @functools.partial(
        pl.kernel,
        mesh=mesh,
        out_type=jax.ShapeDtypeStruct((t, d), table.dtype),
        scratch_types=[
            pltpu.VMEM((per_w,), jnp.int32),
            pltpu.VMEM((per_w,), jnp.int32),
            pltpu.VMEM((per_w, 16), jnp.float32),
            pltpu.VMEM((per_w, 16), jnp.float32),
            pltpu.VMEM((chunk, d), table.dtype),
            pltpu.VMEM((chunk, d), table.dtype),
            pltpu.SemaphoreType.DMA,
        ],
    )
    def combine_kernel(table_hbm, ie_hbm, io_hbm, we_hbm, wo_hbm, out_hbm,
                       ie_v, io_v, we_v, wo_v, a_v, b_v, sem):
        wid = lax.axis_index("s") * _SC_NC + lax.axis_index("c")
        base = wid * per_w
        pltpu.sync_copy(ie_hbm.at[pl.ds(base, per_w)], ie_v)
        pltpu.sync_copy(io_hbm.at[pl.ds(base, per_w)], io_v)
        pltpu.sync_copy(we_hbm.at[pl.ds(base, per_w)], we_v)
        pltpu.sync_copy(wo_hbm.at[pl.ds(base, per_w)], wo_v)
        for c in range(n_chunks):
            pltpu.async_copy(
                table_hbm.at[ie_v.at[pl.ds(c * chunk, chunk)]], a_v, sem
            ).wait()
            pltpu.async_copy(
                table_hbm.at[io_v.at[pl.ds(c * chunk, chunk)]], b_v, sem
            ).wait()

            def row_add(r, carry):
                gr = c * chunk + r
                we = we_v[gr, :]
                wo = wo_v[gr, :]
                for kk in range(d // 16):
                    sl = pl.ds(kk * 16, 16)
                    a_v[r, sl] = a_v[r, sl] * we + b_v[r, sl] * wo
                return carry

            lax.fori_loop(0, chunk, row_add, 0)
            pltpu.sync_copy(a_v, out_hbm.at[pl.ds(base + c * chunk, chunk)])

    return combine_kernel(table, idx_even, idx_odd, w_even, w_odd)


def _gmm_body(te_ref, tv_ref, first_ref, nxt_ref, par_ref, inn_ref,
              hs_ref, wg_hbm, wu_hbm, wd_hbm, out_ref,
              wg_buf, wu_buf, wd_buf, sems):
    t = pl.program_id(0)
    p = par_ref[t]
    e = te_ref[t]

    @pl.when(t == 0)
    def _():
        pltpu.make_async_copy(wg_hbm.at[e], wg_buf.at[0], sems.at[0, 0]).start()
        pltpu.make_async_copy(wu_hbm.at[e], wu_buf.at[0], sems.at[1, 0]).start()
        pltpu.make_async_copy(wd_hbm.at[e], wd_buf.at[0], sems.at[2, 0]).start()

    @pl.when(first_ref[t] == 1)
    def _():
        pltpu.make_async_copy(wg_hbm.at[e], wg_buf.at[p], sems.at[0, p]).wait()
        pltpu.make_async_copy(wu_hbm.at[e], wu_buf.at[p], sems.at[1, p]).wait()
        pltpu.make_async_copy(wd_hbm.at[e], wd_buf.at[p], sems.at[2, p]).wait()

        @pl.when(inn_ref[t] == 1)
        def _():
            ne = nxt_ref[t]
            q = 1 - p
            pltpu.make_async_copy(wg_hbm.at[ne], wg_buf.at[q], sems.at[0, q]).start()
            pltpu.make_async_copy(wu_hbm.at[ne], wu_buf.at[q], sems.at[1, q]).start()
            pltpu.make_async_copy(wd_hbm.at[ne], wd_buf.at[q], sems.at[2, q]).start()

    @pl.when(tv_ref[t] == 1)
    def _():
        rows = hs_ref[...].astype(jnp.bfloat16)
        wg = wg_buf[p].astype(jnp.bfloat16)
        wu = wu_buf[p].astype(jnp.bfloat16)
        wd = wd_buf[p].astype(jnp.bfloat16)
        dn = (((1,), (1,)), ((), ()))
        g = jax.lax.dot_general(rows, wg, dn, preferred_element_type=jnp.float32)
        u = jax.lax.dot_general(rows, wu, dn, preferred_element_type=jnp.float32)
        a = (g * jax.nn.sigmoid(g)) * u
        out_ref[...] = jax.lax.dot_general(a.astype(jnp.bfloat16), wd, dn,
                                           preferred_element_type=jnp.float32)


def kernel(x, Wr, Wg, Wu, Wd):
    bsz, seq, dim = x.shape
    moe_dim = Wg.shape[1]
    h = x.reshape(-1, dim)
    T = h.shape[0]
    P = T * _TOP_K

    topk_idx, topk_w = _router(h, Wr)

    flat_e = topk_idx.reshape(-1)
    onehot = (flat_e[:, None] == jnp.arange(_NUM_EXPERTS, dtype=jnp.int32)[None, :]
              ).astype(jnp.int32)
    cum_oh = jnp.cumsum(onehot, axis=0)
    counts = cum_oh[-1]
    rank = ((cum_oh - onehot) * onehot).sum(axis=1)

    padded = ((counts + _TM - 1) // _TM) * _TM
    pstart = (jnp.cumsum(padded) - padded).astype(jnp.int32)
    dst = pstart[flat_e] + rank

    M_pad = P + _NUM_EXPERTS * _TM
    NT = M_pad // _TM

    dst2 = dst.reshape(T, _TOP_K)
    dst_even = dst2[:, 0].reshape(_SC_NW, T // _SC_NW)
    dst_odd = dst2[:, 1].reshape(_SC_NW, T // _SC_NW)

    hs_pad = _sc_dispatch(h, dst_even, dst_odd, M_pad)

    tiles = (padded // _TM).astype(jnp.int32)
    cum_tiles = jnp.cumsum(tiles)
    t_idx = jnp.arange(NT, dtype=jnp.int32)
    tile_e = jnp.searchsorted(cum_tiles, t_idx, side="right").astype(jnp.int32)
    tile_valid = (tile_e < _NUM_EXPERTS).astype(jnp.int32)
    tile_e = jnp.minimum(tile_e, _NUM_EXPERTS - 1)

    first = jnp.concatenate(
        [jnp.ones((1,), jnp.int32),
         (tile_e[1:] != tile_e[:-1]).astype(jnp.int32)])
    run_idx = jnp.cumsum(first) - 1
    par = (run_idx % 2).astype(jnp.int32)
    nxt_pos = jnp.searchsorted(tile_e, tile_e, side="right").astype(jnp.int32)
    inn = (nxt_pos < NT).astype(jnp.int32)
    nxt = tile_e[jnp.minimum(nxt_pos, NT - 1)]

    grid_spec = pltpu.PrefetchScalarGridSpec(
        num_scalar_prefetch=6,
        grid=(NT,),
        in_specs=[
            pl.BlockSpec((_TM, dim), lambda t, *_: (t, 0)),
            pl.BlockSpec(memory_space=pl.ANY),
            pl.BlockSpec(memory_space=pl.ANY),
            pl.BlockSpec(memory_space=pl.ANY),
        ],
        out_specs=pl.BlockSpec((_TM, dim), lambda t, *_: (t, 0)),
        scratch_shapes=[
            pltpu.VMEM((2, moe_dim, dim), jnp.float32),
            pltpu.VMEM((2, moe_dim, dim), jnp.float32),
            pltpu.VMEM((2, dim, moe_dim), jnp.float32),
            pltpu.SemaphoreType.DMA((3, 2)),
        ],
    )
    out_pad = pl.pallas_call(
        _gmm_body,
        grid_spec=grid_spec,
        out_shape=jax.ShapeDtypeStruct((M_pad, dim), jnp.float32),
        compiler_params=pltpu.CompilerParams(
            dimension_semantics=("arbitrary",)),
    )(tile_e, tile_valid, first, nxt, par, inn, hs_pad, Wg, Wu, Wd)

    we_exp = jnp.broadcast_to(topk_w[:, 0:1], (T, 16))
    wo_exp = jnp.broadcast_to(topk_w[:, 1:2], (T, 16))
    y = _sc_combine_pairs(out_pad, dst2[:, 0], dst2[:, 1], we_exp, wo_exp, 32)
    return y.reshape(bsz, seq, dim)


# speedup vs baseline: 2.2169x; 1.1144x over previous
import functools

import jax
import jax.numpy as jnp
from jax import lax
from jax.experimental import pallas as pl
from jax.experimental.pallas import tpu as pltpu
from jax.experimental.pallas import tpu_sc as plsc

_NUM_EXPERTS = 16
_TOP_K = 2
_TM = 128

_SC_NC = 2
_SC_NS = 16
_SC_NW = _SC_NC * _SC_NS


def _router_body(h_ref, wr_ref, w_ref, dst_ref, counts_ref):
    t = h_ref.shape[0]
    logits = jax.lax.dot_general(
        h_ref[...], wr_ref[...], (((1,), (1,)), ((), ())),
        preferred_element_type=jnp.float32)
    m = jnp.max(logits, axis=1, keepdims=True)
    p = jnp.exp(logits - m)
    probs = p / jnp.sum(p, axis=1, keepdims=True)
    ids = jax.lax.broadcasted_iota(jnp.int32, probs.shape, 1)
    m1 = jnp.max(probs, axis=1, keepdims=True)
    i1 = jnp.min(jnp.where(probs == m1, ids, _NUM_EXPERTS), axis=1, keepdims=True)
    oh1 = (ids == i1).astype(jnp.int32)
    masked = jnp.where(oh1 == 1, -jnp.inf, probs)
    m2 = jnp.max(masked, axis=1, keepdims=True)
    i2 = jnp.min(jnp.where(masked == m2, ids, _NUM_EXPERTS), axis=1, keepdims=True)
    oh2 = (ids == i2).astype(jnp.int32)
    w_ref[...] = jnp.concatenate([m1, m2], axis=1)

    ohb = oh1 + oh2
    c = ohb
    s = 1
    while s < t:
        c = c + jnp.concatenate(
            [jnp.zeros((s, _NUM_EXPERTS), jnp.int32), c[:-s]], axis=0)
        s *= 2
    counts = c[-1:, :]
    cum_excl = c - ohb
    padded = ((counts + _TM - 1) // _TM) * _TM
    eidx = jax.lax.broadcasted_iota(jnp.int32, (_NUM_EXPERTS, _NUM_EXPERTS), 0)
    ejdx = jax.lax.broadcasted_iota(jnp.int32, (_NUM_EXPERTS, _NUM_EXPERTS), 1)
    pstart = jnp.sum(jnp.where(eidx < ejdx, padded.reshape(-1, 1), 0),
                     axis=0, keepdims=True)
    base = pstart + cum_excl
    dst_e = jnp.sum(base * oh1, axis=1, keepdims=True)
    dst_o = jnp.sum((base + oh1) * oh2, axis=1, keepdims=True)
    dst_ref[...] = jnp.concatenate([dst_e, dst_o], axis=1)
    counts_ref[...] = counts


def _router(h, Wr):
    t, dim = h.shape
    return pl.pallas_call(
        _router_body,
        out_shape=(
            jax.ShapeDtypeStruct((t, _TOP_K), jnp.float32),
            jax.ShapeDtypeStruct((t, _TOP_K), jnp.int32),
            jax.ShapeDtypeStruct((1, _NUM_EXPERTS), jnp.int32),
        ),
    )(h, Wr)


def _sc_dispatch(h, dst_even, dst_odd, m_pad):
    t, d = h.shape
    per_w = t // _SC_NW
    mesh = plsc.VectorSubcoreMesh(core_axis_name="c", subcore_axis_name="s")

    @functools.partial(
        pl.kernel,
        mesh=mesh,
        out_type=jax.ShapeDtypeStruct((m_pad, d), h.dtype),
        scratch_types=[
            pltpu.VMEM((per_w,), jnp.int32),
            pltpu.VMEM((per_w,), jnp.int32),
            pltpu.VMEM((per_w, d), h.dtype),
            pltpu.SemaphoreType.DMA,
        ],
    )
    def dispatch_kernel(h_hbm, ie_hbm, io_hbm, out_hbm, ie_v, io_v, rows_v, sem):
        wid = lax.axis_index("s") * _SC_NC + lax.axis_index("c")
        base = wid * per_w
        pltpu.sync_copy(ie_hbm.at[wid], ie_v)
        pltpu.sync_copy(io_hbm.at[wid], io_v)
        pltpu.sync_copy(h_hbm.at[pl.ds(base, per_w)], rows_v)
        pltpu.async_copy(rows_v, out_hbm.at[ie_v], sem).wait()
        pltpu.async_copy(rows_v, out_hbm.at[io_v], sem).wait()

    return dispatch_kernel(h, dst_even, dst_odd)


def _sc_combine_pairs(table, idx_even, idx_odd, w_even, w_odd, chunk):
    n_rows, d = table.shape
    t = idx_even.shape[0]
    per_w = t // _SC_NW
    assert per_w % chunk == 0 and t % (8 * _SC_NW) == 0
    n_chunks = per_w // chunk
    mesh = plsc.VectorSubcoreMesh(core_axis_name="c", subcore_axis_name="s")

    @functools.partial(
        pl.kernel,
        mesh=mesh,
        out_type=jax.ShapeDtypeStruct((t, d), table.dtype),
        scratch_types=[
            pltpu.VMEM((per_w,), jnp.int32),
            pltpu.VMEM((per_w,), jnp.int32),
            pltpu.VMEM((per_w, 16), jnp.float32),
            pltpu.VMEM((per_w, 16), jnp.float32),
            pltpu.VMEM((chunk, d), table.dtype),
            pltpu.VMEM((chunk, d), table.dtype),
            pltpu.SemaphoreType.DMA,
        ],
    )
    def combine_kernel(table_hbm, ie_hbm, io_hbm, we_hbm, wo_hbm, out_hbm,
                       ie_v, io_v, we_v, wo_v, a_v, b_v, sem):
        wid = lax.axis_index("s") * _SC_NC + lax.axis_index("c")
        base = wid * per_w
        pltpu.sync_copy(ie_hbm.at[pl.ds(base, per_w)], ie_v)
        pltpu.sync_copy(io_hbm.at[pl.ds(base, per_w)], io_v)
        pltpu.sync_copy(we_hbm.at[pl.ds(base, per_w)], we_v)
        pltpu.sync_copy(wo_hbm.at[pl.ds(base, per_w)], wo_v)
        for c in range(n_chunks):
            pltpu.async_copy(
                table_hbm.at[ie_v.at[pl.ds(c * chunk, chunk)]], a_v, sem
            ).wait()
            pltpu.async_copy(
                table_hbm.at[io_v.at[pl.ds(c * chunk, chunk)]], b_v, sem
            ).wait()

            def row_add(r, carry):
                gr = c * chunk + r
                we = we_v[gr, :]
                wo = wo_v[gr, :]
                for kk in range(d // 16):
                    sl = pl.ds(kk * 16, 16)
                    a_v[r, sl] = a_v[r, sl] * we + b_v[r, sl] * wo
                return carry

            lax.fori_loop(0, chunk, row_add, 0)
            pltpu.sync_copy(a_v, out_hbm.at[pl.ds(base + c * chunk, chunk)])

    return combine_kernel(table, idx_even, idx_odd, w_even, w_odd)


def _gmm_body(te_ref, tv_ref, first_ref, nxt_ref, par_ref, inn_ref,
              hs_ref, wg_hbm, wu_hbm, wd_hbm, out_ref,
              wg_buf, wu_buf, wd_buf, sems):
    t = pl.program_id(0)
    p = par_ref[t]
    e = te_ref[t]

    @pl.when(t == 0)
    def _():
        pltpu.make_async_copy(wg_hbm.at[e], wg_buf.at[0], sems.at[0, 0]).start()
        pltpu.make_async_copy(wu_hbm.at[e], wu_buf.at[0], sems.at[1, 0]).start()
        pltpu.make_async_copy(wd_hbm.at[e], wd_buf.at[0], sems.at[2, 0]).start()

    @pl.when(first_ref[t] == 1)
    def _():
        pltpu.make_async_copy(wg_hbm.at[e], wg_buf.at[p], sems.at[0, p]).wait()
        pltpu.make_async_copy(wu_hbm.at[e], wu_buf.at[p], sems.at[1, p]).wait()
        pltpu.make_async_copy(wd_hbm.at[e], wd_buf.at[p], sems.at[2, p]).wait()

        @pl.when(inn_ref[t] == 1)
        def _():
            ne = nxt_ref[t]
            q = 1 - p
            pltpu.make_async_copy(wg_hbm.at[ne], wg_buf.at[q], sems.at[0, q]).start()
            pltpu.make_async_copy(wu_hbm.at[ne], wu_buf.at[q], sems.at[1, q]).start()
            pltpu.make_async_copy(wd_hbm.at[ne], wd_buf.at[q], sems.at[2, q]).start()

    @pl.when(tv_ref[t] == 1)
    def _():
        rows = hs_ref[...].astype(jnp.bfloat16)
        wg = wg_buf[p].astype(jnp.bfloat16)
        wu = wu_buf[p].astype(jnp.bfloat16)
        wd = wd_buf[p].astype(jnp.bfloat16)
        dn = (((1,), (1,)), ((), ()))
        g = jax.lax.dot_general(rows, wg, dn, preferred_element_type=jnp.float32)
        u = jax.lax.dot_general(rows, wu, dn, preferred_element_type=jnp.float32)
        a = (g * jax.nn.sigmoid(g)) * u
        out_ref[...] = jax.lax.dot_general(a.astype(jnp.bfloat16), wd, dn,
                                           preferred_element_type=jnp.float32)


def kernel(x, Wr, Wg, Wu, Wd):
    bsz, seq, dim = x.shape
    moe_dim = Wg.shape[1]
    h = x.reshape(-1, dim)
    T = h.shape[0]
    P = T * _TOP_K

    topk_w, dst2, counts2 = _router(h, Wr)
    counts = counts2[0]

    padded = ((counts + _TM - 1) // _TM) * _TM

    M_pad = P + _NUM_EXPERTS * _TM
    NT = M_pad // _TM

    dst_even = dst2[:, 0].reshape(_SC_NW, T // _SC_NW)
    dst_odd = dst2[:, 1].reshape(_SC_NW, T // _SC_NW)

    hs_pad = _sc_dispatch(h, dst_even, dst_odd, M_pad)

    tiles = (padded // _TM).astype(jnp.int32)
    cum_tiles = jnp.cumsum(tiles)
    t_idx = jnp.arange(NT, dtype=jnp.int32)
    tile_e = jnp.searchsorted(cum_tiles, t_idx, side="right").astype(jnp.int32)
    tile_valid = (tile_e < _NUM_EXPERTS).astype(jnp.int32)
    tile_e = jnp.minimum(tile_e, _NUM_EXPERTS - 1)

    first = jnp.concatenate(
        [jnp.ones((1,), jnp.int32),
         (tile_e[1:] != tile_e[:-1]).astype(jnp.int32)])
    run_idx = jnp.cumsum(first) - 1
    par = (run_idx % 2).astype(jnp.int32)
    nxt_pos = jnp.searchsorted(tile_e, tile_e, side="right").astype(jnp.int32)
    inn = (nxt_pos < NT).astype(jnp.int32)
    nxt = tile_e[jnp.minimum(nxt_pos, NT - 1)]

    grid_spec = pltpu.PrefetchScalarGridSpec(
        num_scalar_prefetch=6,
        grid=(NT,),
        in_specs=[
            pl.BlockSpec((_TM, dim), lambda t, *_: (t, 0)),
            pl.BlockSpec(memory_space=pl.ANY),
            pl.BlockSpec(memory_space=pl.ANY),
            pl.BlockSpec(memory_space=pl.ANY),
        ],
        out_specs=pl.BlockSpec((_TM, dim), lambda t, *_: (t, 0)),
        scratch_shapes=[
            pltpu.VMEM((2, moe_dim, dim), jnp.float32),
            pltpu.VMEM((2, moe_dim, dim), jnp.float32),
            pltpu.VMEM((2, dim, moe_dim), jnp.float32),
            pltpu.SemaphoreType.DMA((3, 2)),
        ],
    )
    out_pad = pl.pallas_call(
        _gmm_body,
        grid_spec=grid_spec,
        out_shape=jax.ShapeDtypeStruct((M_pad, dim), jnp.float32),
        compiler_params=pltpu.CompilerParams(
            dimension_semantics=("arbitrary",)),
    )(tile_e, tile_valid, first, nxt, par, inn, hs_pad, Wg, Wu, Wd)

    we_exp = jnp.broadcast_to(topk_w[:, 0:1], (T, 16))
    wo_exp = jnp.broadcast_to(topk_w[:, 1:2], (T, 16))
    y = _sc_combine_pairs(out_pad, dst2[:, 0], dst2[:, 1], we_exp, wo_exp, 32)
    return y.reshape(bsz, seq, dim)
